# Initial kernel scaffold; baseline (speedup 1.0000x reference)
import jax
import jax.numpy as jnp
from jax.experimental import pallas as pl


def kernel(x, member_atoms, member_domains, W, b):
    raise NotImplementedError("write your pallas kernel here")


# speedup vs baseline: 12.4765x; 12.4765x over previous
import functools

import jax
import jax.numpy as jnp
from jax import lax
from jax.experimental import pallas as pl
from jax.experimental.pallas import tpu as pltpu
from jax.experimental.pallas import tpu_sc as plsc

N_NODES = 10000
N_MEMBERS = 320000
D = 128

NC = 2
NS = 16
NW = NC * NS
M_PER_W = N_MEMBERS // NW
K = 80
NCHUNK = M_PER_W // K
ROWS_MAIN = 624
ROWS_TAIL = N_NODES - NS * ROWS_MAIN

_mesh = plsc.VectorSubcoreMesh(core_axis_name="c", subcore_axis_name="s")


@functools.partial(
    pl.kernel,
    out_type=jax.ShapeDtypeStruct((NC, N_NODES, D), jnp.float32),
    mesh=_mesh,
    scratch_types=[
        pltpu.VMEM((M_PER_W,), jnp.int32),
        pltpu.VMEM((M_PER_W,), jnp.int32),
        pltpu.VMEM((K,), jnp.int32),
        pltpu.VMEM((K, D), jnp.float32),
        pltpu.VMEM((K, D), jnp.float32),
        pltpu.VMEM_SHARED((N_NODES, D), jnp.float32),
        pltpu.SemaphoreType.DMA,
        pltpu.SemaphoreType.DMA,
    ],
)
def _transfer_sc(x_hbm, atoms_hbm, doms_hbm, zeros_hbm, out_hbm,
                 atoms_v, doms_v, domk_v, rows0_v, rows1_v, f_sh, sem0, sem1):
    cid = lax.axis_index("c")
    sid = lax.axis_index("s")
    wid = sid * NC + cid
    base = wid * M_PER_W

    row0 = pl.multiple_of(sid * ROWS_MAIN, 8)
    pltpu.sync_copy(zeros_hbm.at[pl.ds(row0, ROWS_MAIN)],
                    f_sh.at[pl.ds(row0, ROWS_MAIN)])

    @pl.when(sid == NS - 1)
    def _():
        pltpu.sync_copy(zeros_hbm.at[pl.ds(NS * ROWS_MAIN, ROWS_TAIL)],
                        f_sh.at[pl.ds(NS * ROWS_MAIN, ROWS_TAIL)])

    pltpu.sync_copy(atoms_hbm.at[pl.ds(base, M_PER_W)], atoms_v)
    pltpu.sync_copy(doms_hbm.at[pl.ds(base, M_PER_W)], doms_v)
    plsc.subcore_barrier()

    bufs = (rows0_v, rows1_v)
    sems = (sem0, sem1)

    pltpu.async_copy(x_hbm.at[atoms_v.at[pl.ds(0, K)]], rows0_v, sem0)

    def body(j, _):
        def run(j, cur, nxt, cur_sem):
            @pl.when(j < NCHUNK - 1)
            def _():
                nxt_idx = atoms_v.at[pl.ds((j + 1) * K, K)]
                pltpu.async_copy(x_hbm.at[nxt_idx], nxt,
                                 sems[0] if nxt is bufs[0] else sems[1])
            for i in range(K // 16):
                domk_v[pl.ds(i * 16, 16)] = doms_v[pl.ds(j * K + i * 16, 16)]
            pltpu.make_async_copy(
                x_hbm.at[atoms_v.at[pl.ds(j * K, K)]], cur, cur_sem).wait()
            pltpu.sync_copy(cur, f_sh.at[domk_v], add=True)

        @pl.when(j % 2 == 0)
        def _():
            run(j, bufs[0], bufs[1], sems[0])

        @pl.when(j % 2 == 1)
        def _():
            run(j, bufs[1], bufs[0], sems[1])
        return ()

    lax.fori_loop(0, NCHUNK, body, ())

    plsc.subcore_barrier()
    pltpu.sync_copy(f_sh.at[pl.ds(row0, ROWS_MAIN)],
                    out_hbm.at[cid].at[pl.ds(row0, ROWS_MAIN)])

    @pl.when(sid == NS - 1)
    def _():
        pltpu.sync_copy(f_sh.at[pl.ds(NS * ROWS_MAIN, ROWS_TAIL)],
                        out_hbm.at[cid].at[pl.ds(NS * ROWS_MAIN, ROWS_TAIL)])


_BLK = 1000


def _mm_body(f2_ref, w_ref, b_ref, o_ref):
    f = f2_ref[0] + f2_ref[1]
    o_ref[...] = (
        jnp.dot(f, w_ref[...], preferred_element_type=jnp.float32)
        + b_ref[...]
    )


@jax.jit
def _linear_tc(partials, W, b2):
    return pl.pallas_call(
        _mm_body,
        grid=(N_NODES // _BLK,),
        in_specs=[
            pl.BlockSpec((NC, _BLK, D), lambda i: (0, i, 0)),
            pl.BlockSpec((D, D), lambda i: (0, 0)),
            pl.BlockSpec((1, D), lambda i: (0, 0)),
        ],
        out_specs=pl.BlockSpec((_BLK, D), lambda i: (i, 0)),
        out_shape=jax.ShapeDtypeStruct((N_NODES, D), jnp.float32),
    )(partials, W, b2)


def kernel(x, member_atoms, member_domains, W, b):
    atoms = member_atoms.astype(jnp.int32)
    doms = member_domains.astype(jnp.int32)
    zeros = jnp.zeros((N_NODES, D), jnp.float32)
    partials = _transfer_sc(x, atoms, doms, zeros)
    return _linear_tc(partials, W, b.reshape(1, D))


# speedup vs baseline: 14.2062x; 1.1386x over previous
import functools

import jax
import jax.numpy as jnp
from jax import lax
from jax.experimental import pallas as pl
from jax.experimental.pallas import tpu as pltpu
from jax.experimental.pallas import tpu_sc as plsc

N_NODES = 10000
N_MEMBERS = 320000
D = 128

NC = 2
NS = 16
NW = NC * NS
M_PER_W = N_MEMBERS // NW
K = 80
NCHUNK = M_PER_W // K
NBUF = 4
NIDX = 8
LOOP_LAPS = NCHUNK // NIDX
LOOP_CHUNKS = LOOP_LAPS * NIDX
ROWS_MAIN = 624
ROWS_TAIL = N_NODES - NS * ROWS_MAIN

_mesh = plsc.VectorSubcoreMesh(core_axis_name="c", subcore_axis_name="s")


@functools.partial(
    pl.kernel,
    out_type=jax.ShapeDtypeStruct((NC, N_NODES, D), jnp.float32),
    mesh=_mesh,
    scratch_types=(
        [pltpu.VMEM((K, D), jnp.float32) for _ in range(NBUF)]
        + [pltpu.VMEM((K,), jnp.int32) for _ in range(NIDX)]
        + [pltpu.VMEM((K,), jnp.int32) for _ in range(NIDX)]
        + [pltpu.VMEM_SHARED((N_NODES, D), jnp.float32)]
        + [pltpu.SemaphoreType.DMA for _ in range(NBUF + NIDX)]
    ),
)
def _transfer_sc(x_hbm, atoms_hbm, doms_hbm, zeros_hbm, out_hbm, *rest):
    rows = rest[0:NBUF]
    aidx = rest[NBUF:NBUF + NIDX]
    didx = rest[NBUF + NIDX:NBUF + 2 * NIDX]
    f_sh = rest[NBUF + 2 * NIDX]
    sems = rest[NBUF + 2 * NIDX + 1:]
    sem_g = sems[0:NBUF]
    sem_i = sems[NBUF:NBUF + NIDX]

    cid = lax.axis_index("c")
    sid = lax.axis_index("s")
    wid = sid * NC + cid
    base = wid * M_PER_W

    for j in range(NIDX):
        off = base + j * K
        pltpu.async_copy(atoms_hbm.at[pl.ds(off, K)], aidx[j], sem_i[j])
        pltpu.async_copy(doms_hbm.at[pl.ds(off, K)], didx[j], sem_i[j])

    row0 = pl.multiple_of(sid * ROWS_MAIN, 8)
    pltpu.sync_copy(zeros_hbm.at[pl.ds(row0, ROWS_MAIN)],
                    f_sh.at[pl.ds(row0, ROWS_MAIN)])

    @pl.when(sid == NS - 1)
    def _():
        pltpu.sync_copy(zeros_hbm.at[pl.ds(NS * ROWS_MAIN, ROWS_TAIL)],
                        f_sh.at[pl.ds(NS * ROWS_MAIN, ROWS_TAIL)])

    plsc.subcore_barrier()

    for j in range(NBUF):
        off = base + j * K
        pltpu.make_async_copy(atoms_hbm.at[pl.ds(off, K)], aidx[j],
                              sem_i[j]).wait()
        pltpu.make_async_copy(doms_hbm.at[pl.ds(off, K)], didx[j],
                              sem_i[j]).wait()
        pltpu.async_copy(x_hbm.at[aidx[j]], rows[j], sem_g[j])

    def lap(g, _):
        for k in range(NIDX):
            b = k % NBUF
            bp = (b + NBUF - 1) % NBUF
            kp = (k + NIDX - 1) % NIDX
            kg = (k + NBUF - 1) % NIDX
            c = g * NIDX + k

            def head(b=b, bp=bp, kp=kp, kg=kg, c=c, k=k):
                goff = base + (c + NBUF - 1) * K
                pltpu.make_async_copy(atoms_hbm.at[pl.ds(goff, K)],
                                      aidx[kg], sem_i[kg]).wait()
                pltpu.make_async_copy(doms_hbm.at[pl.ds(goff, K)],
                                      didx[kg], sem_i[kg]).wait()
                pltpu.async_copy(x_hbm.at[aidx[kg]], rows[bp], sem_g[bp])

                def pref(kp=kp, c=c):
                    poff = base + (c + NIDX - 1) * K
                    pltpu.async_copy(atoms_hbm.at[pl.ds(poff, K)],
                                     aidx[kp], sem_i[kp])
                    pltpu.async_copy(doms_hbm.at[pl.ds(poff, K)],
                                     didx[kp], sem_i[kp])

                if k >= NIDX - 2:
                    pl.when(g < LOOP_LAPS - 1)(pref)
                else:
                    pref()

            if k == 0:
                pl.when(g > 0)(head)
            else:
                head()

            pltpu.make_async_copy(x_hbm.at[aidx[k]], rows[b],
                                  sem_g[b]).wait()
            pltpu.sync_copy(rows[b], f_sh.at[didx[k]], add=True)
        return ()

    lax.fori_loop(0, LOOP_LAPS, lap, ())

    for c in range(LOOP_CHUNKS, NCHUNK):
        k = c % NIDX
        b = c % NBUF
        bp = (b + NBUF - 1) % NBUF
        gc = c + NBUF - 1
        if gc < NCHUNK:
            kg = gc % NIDX
            goff = base + gc * K
            pltpu.make_async_copy(atoms_hbm.at[pl.ds(goff, K)],
                                  aidx[kg], sem_i[kg]).wait()
            pltpu.make_async_copy(doms_hbm.at[pl.ds(goff, K)],
                                  didx[kg], sem_i[kg]).wait()
            pltpu.async_copy(x_hbm.at[aidx[kg]], rows[bp], sem_g[bp])
        pltpu.make_async_copy(x_hbm.at[aidx[k]], rows[b], sem_g[b]).wait()
        pltpu.sync_copy(rows[b], f_sh.at[didx[k]], add=True)

    plsc.subcore_barrier()
    pltpu.sync_copy(f_sh.at[pl.ds(row0, ROWS_MAIN)],
                    out_hbm.at[cid].at[pl.ds(row0, ROWS_MAIN)])

    @pl.when(sid == NS - 1)
    def _():
        pltpu.sync_copy(f_sh.at[pl.ds(NS * ROWS_MAIN, ROWS_TAIL)],
                        out_hbm.at[cid].at[pl.ds(NS * ROWS_MAIN, ROWS_TAIL)])


_BLK = 1000


def _mm_body(f2_ref, w_ref, b_ref, o_ref):
    f = f2_ref[0] + f2_ref[1]
    o_ref[...] = (
        jnp.dot(f, w_ref[...], preferred_element_type=jnp.float32)
        + b_ref[...]
    )


@jax.jit
def _linear_tc(partials, W, b2):
    return pl.pallas_call(
        _mm_body,
        grid=(N_NODES // _BLK,),
        in_specs=[
            pl.BlockSpec((NC, _BLK, D), lambda i: (0, i, 0)),
            pl.BlockSpec((D, D), lambda i: (0, 0)),
            pl.BlockSpec((1, D), lambda i: (0, 0)),
        ],
        out_specs=pl.BlockSpec((_BLK, D), lambda i: (i, 0)),
        out_shape=jax.ShapeDtypeStruct((N_NODES, D), jnp.float32),
    )(partials, W, b2)


def kernel(x, member_atoms, member_domains, W, b):
    atoms = member_atoms.astype(jnp.int32)
    doms = member_domains.astype(jnp.int32)
    zeros = jnp.zeros((N_NODES, D), jnp.float32)
    partials = _transfer_sc(x, atoms, doms, zeros)
    return _linear_tc(partials, W, b.reshape(1, D))
